# baseline (speedup 1.0000x reference)
import functools

import jax
import jax.numpy as jnp
from jax import lax
from jax.experimental import pallas as pl
from jax.experimental.pallas import tpu as pltpu
from jax.experimental.pallas import tpu_sc as plsc


def _contract_body(basis_ref, cmat_ref, w_ref):
    w_ref[...] = jnp.dot(
        basis_ref[...], cmat_ref[...], preferred_element_type=jnp.float32
    )


def _materialize_weight(basis4, cmat, rows_per_block):
    n4, de4 = basis4.shape
    dp = cmat.shape[1]
    return pl.pallas_call(
        _contract_body,
        grid=(n4 // rows_per_block,),
        in_specs=[
            pl.BlockSpec((rows_per_block, de4), lambda i: (i, 0)),
            pl.BlockSpec((de4, dp), lambda i: (0, 0)),
        ],
        out_specs=pl.BlockSpec((rows_per_block, dp), lambda i: (i, 0)),
        out_shape=jax.ShapeDtypeStruct((n4, dp), jnp.float32),
    )(basis4, cmat)


_CHUNK = 128


def _sc_gather(table, flat_idx, d):
    info = plsc.get_sparse_core_info()
    nc, ns = info.num_cores, info.num_subcores
    nw = nc * ns
    b = flat_idx.shape[0]
    b_per_w = b // nw
    n_ch = b_per_w // _CHUNK
    n_vec = b_per_w // 16

    mesh = plsc.VectorSubcoreMesh(core_axis_name="c", subcore_axis_name="s")

    @functools.partial(
        pl.kernel,
        mesh=mesh,
        out_type=jax.ShapeDtypeStruct((b * d,), jnp.float32),
        scratch_types=[
            pltpu.VMEM((b_per_w,), jnp.int32),
            pltpu.VMEM((b_per_w,), jnp.int32),
            pltpu.VMEM((b_per_w,), jnp.int32),
            pltpu.VMEM((_CHUNK, 128), jnp.float32),
            pltpu.VMEM((b_per_w * d,), jnp.float32),
            pltpu.SemaphoreType.DMA,
        ],
        compiler_params=pltpu.CompilerParams(
            use_tc_tiling_on_sc=False, needs_layout_passes=False
        ),
    )
    def k(idx_hbm, table_hbm, out_hbm, idx_v, idx4_v, idxm_v, buf_v, rows_v, sem):
        wid = lax.axis_index("s") * nc + lax.axis_index("c")
        base = wid * b_per_w
        pltpu.sync_copy(idx_hbm.at[pl.ds(base, b_per_w)], idx_v)

        def prep(i, _):
            v = idx_v[pl.ds(i * 16, 16)]
            idx4_v[pl.ds(i * 16, 16)] = lax.shift_right_logical(v, 2)
            idxm_v[pl.ds(i * 16, 16)] = lax.bitwise_and(v, 3)
            return _

        lax.fori_loop(0, n_vec, prep, None)

        lanes = lax.iota(jnp.int32, 16)

        def chunk(c, _):
            off = c * _CHUNK
            pltpu.async_copy(
                table_hbm.at[idx4_v.at[pl.ds(off, _CHUNK)]], buf_v, sem
            ).wait()

            def grp(g, _2):
                lrow = g * 16 + lanes
                row = off + lrow
                colb = idxm_v[pl.ds(off + g * 16, 16)] * d
                dstb = row * d
                for c2 in range(d):
                    v = plsc.load_gather(buf_v, [lrow, colb + c2])
                    plsc.store_scatter(rows_v, [dstb + c2], v)
                return _2

            lax.fori_loop(0, _CHUNK // 16, grp, None)
            return _

        lax.fori_loop(0, n_ch, chunk, None)
        pltpu.sync_copy(rows_v, out_hbm.at[pl.ds(base * d, b_per_w * d)])

    return k(flat_idx, table)


def kernel(indices, coef, basis):
    n, d, e = basis.shape
    basis4 = basis.reshape(n // 4, 4 * d * e)
    de4 = 4 * d * e
    dp = 4 * d
    q = jnp.arange(de4)
    c = jnp.arange(dp)
    mask = ((q[:, None] // (d * e)) == (c[None, :] // d)) & (
        ((q[:, None] % (d * e)) // e) == (c[None, :] % d)
    )
    cmat = jnp.where(mask, coef[q % e][:, None], 0.0).astype(jnp.float32)
    table = _materialize_weight(basis4, cmat, rows_per_block=1000)
    flat_idx = indices.reshape(-1).astype(jnp.int32)
    out_flat = _sc_gather(table, flat_idx, d)
    return out_flat.reshape(indices.shape[0], indices.shape[1], d)


# speedup vs baseline: 6.4090x; 6.4090x over previous
import functools

import jax
import jax.numpy as jnp
from jax import lax
from jax.experimental import pallas as pl
from jax.experimental.pallas import tpu as pltpu
from jax.experimental.pallas import tpu_sc as plsc


def _contract_body(basis_ref, cmat_ref, w_ref):
    w_ref[...] = jnp.dot(
        basis_ref[...], cmat_ref[...], preferred_element_type=jnp.float32
    )


def _materialize_weight(basis2, cmat, rows_per_block):
    n, de = basis2.shape
    d = cmat.shape[1]
    return pl.pallas_call(
        _contract_body,
        grid=(n // rows_per_block,),
        in_specs=[
            pl.BlockSpec((rows_per_block, de), lambda i: (i, 0)),
            pl.BlockSpec((de, d), lambda i: (0, 0)),
        ],
        out_specs=pl.BlockSpec((rows_per_block, d), lambda i: (i, 0)),
        out_shape=jax.ShapeDtypeStruct((n, d), jnp.float32),
    )(basis2, cmat)


_CHUNK = 128


def _sc_gather(weight, flat_idx):
    info = plsc.get_sparse_core_info()
    nc, ns = info.num_cores, info.num_subcores
    nw = nc * ns
    b = flat_idx.shape[0]
    d = weight.shape[1]
    b_per_w = b // nw
    n_ch = b_per_w // _CHUNK

    mesh = plsc.VectorSubcoreMesh(core_axis_name="c", subcore_axis_name="s")

    @functools.partial(
        pl.kernel,
        mesh=mesh,
        out_type=jax.ShapeDtypeStruct((b, d), jnp.float32),
        scratch_types=[
            pltpu.VMEM((b_per_w,), jnp.int32),
            pltpu.VMEM((b_per_w, d), jnp.float32),
            pltpu.SemaphoreType.DMA,
        ],
        compiler_params=pltpu.CompilerParams(use_tc_tiling_on_sc=False),
    )
    def k(idx_hbm, table_hbm, out_hbm, idx_v, rows_v, sem):
        wid = lax.axis_index("s") * nc + lax.axis_index("c")
        base = wid * b_per_w
        pltpu.sync_copy(idx_hbm.at[pl.ds(base, b_per_w)], idx_v)

        def descr(c):
            off = c * _CHUNK
            return pltpu.make_async_copy(
                table_hbm.at[idx_v.at[pl.ds(off, _CHUNK)]],
                rows_v.at[pl.ds(off, _CHUNK)],
                sem,
            )

        def fire(c, _):
            descr(c).start()
            return _

        def drain(c, _):
            descr(c).wait()
            return _

        lax.fori_loop(0, n_ch, fire, None)
        lax.fori_loop(0, n_ch, drain, None)
        pltpu.sync_copy(rows_v, out_hbm.at[pl.ds(base, b_per_w)])

    return k(flat_idx, weight)


def kernel(indices, coef, basis):
    n, d, e = basis.shape
    basis2 = basis.reshape(n, d * e)
    cmat = (jnp.eye(d, dtype=coef.dtype)[:, None, :] * coef[None, :, None]).reshape(
        d * e, d
    )
    weight = _materialize_weight(basis2, cmat, rows_per_block=10000)
    flat_idx = indices.reshape(-1).astype(jnp.int32)
    out_flat = _sc_gather(weight, flat_idx)
    return out_flat.reshape(indices.shape[0], indices.shape[1], d)
